# baseline (speedup 1.0000x reference)
import dataclasses
import functools

import jax
import jax.numpy as jnp
from jax import lax
from jax.experimental import pallas as pl
from jax.experimental.pallas import tpu as pltpu
from jax.experimental.pallas import tpu_sc as plsc

_NC, _NS, _L = 2, 16, 16
_NW = _NC * _NS

_SC_CP = pltpu.CompilerParams()
if "needs_layout_passes" in pltpu.CompilerParams.__dataclass_fields__:
    _SC_CP = dataclasses.replace(_SC_CP, needs_layout_passes=False)


def _mm_kernel(h_ref, w_ref, o_ref):
    o_ref[...] = jnp.dot(h_ref[...], w_ref[...], preferred_element_type=jnp.float32)


def _mm(h, w):
    n, d_in = h.shape
    d_out = w.shape[1]
    blk = min(n, 2000)
    return pl.pallas_call(
        _mm_kernel,
        grid=(n // blk,),
        in_specs=[
            pl.BlockSpec((blk, d_in), lambda i: (i, 0)),
            pl.BlockSpec((d_in, d_out), lambda i: (0, 0)),
        ],
        out_specs=pl.BlockSpec((blk, d_out), lambda i: (i, 0)),
        out_shape=jax.ShapeDtypeStruct((n, d_out), jnp.float32),
    )(h, w)


def _bn_relu_kernel(z_ref, g_ref, be_ref, o_ref, *, relu):
    z = z_ref[...]
    n = z.shape[0]
    m = jnp.sum(z, axis=0, keepdims=True) / n
    zc = z - m
    v = jnp.sum(zc * zc, axis=0, keepdims=True) / n
    out = g_ref[...] * zc * jax.lax.rsqrt(v + 1e-5) + be_ref[...]
    if relu:
        out = jnp.maximum(out, 0.0)
    o_ref[...] = out


def _bn_relu(z, g, be, relu):
    n, d = z.shape
    blk = 128
    return pl.pallas_call(
        functools.partial(_bn_relu_kernel, relu=relu),
        grid=(d // blk,),
        in_specs=[
            pl.BlockSpec((n, blk), lambda i: (0, i)),
            pl.BlockSpec((1, blk), lambda i: (0, i)),
            pl.BlockSpec((1, blk), lambda i: (0, i)),
        ],
        out_specs=pl.BlockSpec((n, blk), lambda i: (0, i)),
        out_shape=jax.ShapeDtypeStruct((n, d), jnp.float32),
    )(z, g.reshape(1, -1), be.reshape(1, -1))


def _deg_sc(colp1, ewp1, n_pad):
    e_pad = colp1.shape[0]
    ew_t = e_pad // _NW
    nps = n_pad // _NS
    mesh = plsc.VectorSubcoreMesh(core_axis_name="c", subcore_axis_name="s")

    @functools.partial(
        pl.kernel,
        mesh=mesh,
        out_type=jax.ShapeDtypeStruct((_NC, n_pad), jnp.float32),
        compiler_params=_SC_CP,
        scratch_types=[
            pltpu.VMEM((ew_t,), jnp.int32),
            pltpu.VMEM((ew_t,), jnp.float32),
            pltpu.VMEM((n_pad,), jnp.float32),
            pltpu.VMEM((_NS * nps,), jnp.float32),
            pltpu.VMEM((nps,), jnp.float32),
            pltpu.VMEM_SHARED((_NS, n_pad), jnp.float32),
        ],
    )
    def deg_kernel(col_hbm, ew_hbm, out_hbm, colv, ewv, acc, rbuf, obuf, part):
        ci = lax.axis_index("c")
        si = lax.axis_index("s")
        wid = ci * _NS + si

        @pl.loop(0, n_pad // _L)
        def _(z):
            acc[pl.ds(z * _L, _L)] = jnp.zeros((_L,), jnp.float32)

        pltpu.sync_copy(col_hbm.at[pl.ds(wid * ew_t, ew_t)], colv)
        pltpu.sync_copy(ew_hbm.at[pl.ds(wid * ew_t, ew_t)], ewv)

        @pl.loop(0, ew_t // _L)
        def _(t):
            idx = colv[pl.ds(t * _L, _L)]
            w = ewv[pl.ds(t * _L, _L)]
            plsc.addupdate_scatter(acc, [idx], w)

        pltpu.sync_copy(acc, part.at[si])
        plsc.subcore_barrier()

        for l in range(_NS):
            pltpu.sync_copy(part.at[l, pl.ds(si * nps, nps)],
                            rbuf.at[pl.ds(l * nps, nps)])

        @pl.loop(0, nps // _L)
        def _(m):
            s = jnp.zeros((_L,), jnp.float32)
            for l in range(_NS):
                s = s + rbuf[pl.ds(l * nps + m * _L, _L)]
            obuf[pl.ds(m * _L, _L)] = s

        pltpu.sync_copy(obuf, out_hbm.at[ci, pl.ds(si * nps, nps)])

    return deg_kernel(colp1, ewp1)


def _agg_sc(y3, rowp2, colp2, ewp1, n_pad):
    c_chunks = y3.shape[0]
    cw = y3.shape[2]
    e_pad = ewp1.shape[0]
    blocks_w = e_pad // 128 // _NW
    ew_t = blocks_w * 128
    nps = n_pad // _NS
    nzb = 16
    mesh = plsc.VectorSubcoreMesh(core_axis_name="c", subcore_axis_name="s")

    @functools.partial(
        pl.kernel,
        mesh=mesh,
        out_type=jax.ShapeDtypeStruct((_NC, c_chunks, n_pad, cw), jnp.float32),
        compiler_params=_SC_CP,
        scratch_types=[
            pltpu.VMEM((8, 128), jnp.int32),
            pltpu.VMEM((8, 128), jnp.int32),
            pltpu.VMEM((1024,), jnp.float32),
            pltpu.VMEM((128, cw), jnp.float32),
            pltpu.VMEM((128, cw), jnp.float32),
            pltpu.VMEM((nzb, cw), jnp.float32),
            pltpu.VMEM_SHARED((n_pad, cw), jnp.float32),
        ],
    )
    def agg_kernel(y_hbm, row_hbm, col_hbm, ew_hbm, out_hbm,
                   rowv, colv, ewv, g, g2, zbuf, acc):
        ci = lax.axis_index("c")
        si = lax.axis_index("s")
        wid = ci * _NS + si
        groups = blocks_w // 8

        @pl.loop(0, nzb)
        def _(r):
            for k in range(cw // _L):
                zbuf[r, pl.ds(k * _L, _L)] = jnp.zeros((_L,), jnp.float32)

        for c in range(c_chunks):
            for b in range(nps // nzb):
                pltpu.sync_copy(zbuf, acc.at[pl.ds(si * nps + b * nzb, nzb)])
            plsc.subcore_barrier()

            @pl.loop(0, groups)
            def _(jg):
                pltpu.sync_copy(
                    row_hbm.at[pl.ds(wid * blocks_w + jg * 8, 8)], rowv)
                pltpu.sync_copy(
                    col_hbm.at[pl.ds(wid * blocks_w + jg * 8, 8)], colv)
                pltpu.sync_copy(
                    ew_hbm.at[pl.ds(wid * ew_t + jg * 1024, 1024)], ewv)

                @pl.loop(0, 8)
                def _(jj):
                    pltpu.sync_copy(y_hbm.at[c].at[rowv.at[jj]], g)

                    @plsc.parallel_loop(0, 128, 1, unroll=4)
                    def _(e):
                        wv = ewv[pl.ds(jj * 128 + (e // _L) * _L, _L)]
                        spl = wv.at[jnp.full((_L,), e % _L, jnp.int32)].get(
                            mode="promise_in_bounds")
                        for k in range(cw // _L):
                            g2[e, pl.ds(k * _L, _L)] = (
                                g[e, pl.ds(k * _L, _L)] * spl)

                    pltpu.sync_copy(g2, acc.at[colv.at[jj]], add=True)

            plsc.subcore_barrier()
            pltpu.sync_copy(acc.at[pl.ds(si * nps, nps)],
                            out_hbm.at[ci, c, pl.ds(si * nps, nps)])
            plsc.subcore_barrier()

    return agg_kernel(y3, rowp2, colp2, ewp1)


def _dis_kernel(degp_ref, o_ref):
    deg = degp_ref[0] + degp_ref[1] + 1.0
    o_ref[...] = jax.lax.rsqrt(deg)


def _dis(degp):
    return pl.pallas_call(
        _dis_kernel,
        out_shape=jax.ShapeDtypeStruct(degp.shape[1:], jnp.float32),
    )(degp)


def kernel(x, edge_index, edge_weight, batch, params):
    n = x.shape[0]
    e = edge_weight.shape[0]
    epad = ((e + _NW * 1024 - 1) // (_NW * 1024)) * (_NW * 1024)
    npad = epad - e
    pad_idx = (jnp.arange(npad, dtype=jnp.int32) * 97) % n

    row = jnp.concatenate([edge_index[0], pad_idx])
    col = jnp.concatenate([edge_index[1], pad_idx])
    ew = jnp.concatenate([edge_weight, jnp.zeros((npad,), jnp.float32)])
    n_pad = 10240

    degp = _deg_sc(col, ew, n_pad)
    dis = _dis(degp)[:n]
    disv = dis[:, None]

    rowp2 = row.reshape(-1, 128)
    colp2 = col.reshape(-1, 128)

    cw = 128
    h = x
    for i in range(1, 6):
        xw = _mm(h, params[f"w{i}"])
        y = xw * disv
        d = y.shape[1]
        c_chunks = d // cw
        y3 = jnp.pad(y, ((0, n_pad - n), (0, 0))) \
            .reshape(n_pad, c_chunks, cw).transpose(1, 0, 2)
        aggp = _agg_sc(y3, rowp2, colp2, ew, n_pad)
        agg = ((aggp[0] + aggp[1]).transpose(1, 0, 2).reshape(n_pad, d)[:n]
               * disv + (disv * disv) * xw)
        z = agg + params[f"b{i}"]
        h = _bn_relu(z, params[f"g{i}"], params[f"be{i}"], relu=(i < 5))

    sums = jax.ops.segment_sum(h, batch, num_segments=64)
    cnt = jax.ops.segment_sum(jnp.ones((n,), h.dtype), batch, num_segments=64)
    pooled = sums / jnp.maximum(cnt, 1.0)[:, None]
    pooled = jax.nn.relu(pooled)
    return _mm(pooled, params["w_fc"]) + params["b_fc"]


# speedup vs baseline: 1.4278x; 1.4278x over previous
import dataclasses
import functools

import jax
import jax.numpy as jnp
from jax import lax
from jax.experimental import pallas as pl
from jax.experimental.pallas import tpu as pltpu
from jax.experimental.pallas import tpu_sc as plsc

_NC, _NS, _L = 2, 16, 16
_NW = _NC * _NS

_SC_CP = pltpu.CompilerParams()
if "needs_layout_passes" in pltpu.CompilerParams.__dataclass_fields__:
    _SC_CP = dataclasses.replace(_SC_CP, needs_layout_passes=False)


def _mm_kernel(h_ref, w_ref, o_ref):
    o_ref[...] = jnp.dot(h_ref[...], w_ref[...], preferred_element_type=jnp.float32)


def _mm(h, w):
    n, d_in = h.shape
    d_out = w.shape[1]
    blk = min(n, 2000)
    return pl.pallas_call(
        _mm_kernel,
        grid=(n // blk,),
        in_specs=[
            pl.BlockSpec((blk, d_in), lambda i: (i, 0)),
            pl.BlockSpec((d_in, d_out), lambda i: (0, 0)),
        ],
        out_specs=pl.BlockSpec((blk, d_out), lambda i: (i, 0)),
        out_shape=jax.ShapeDtypeStruct((n, d_out), jnp.float32),
    )(h, w)


def _bn_relu_kernel(z_ref, g_ref, be_ref, o_ref, *, relu):
    z = z_ref[...]
    n = z.shape[0]
    m = jnp.sum(z, axis=0, keepdims=True) / n
    zc = z - m
    v = jnp.sum(zc * zc, axis=0, keepdims=True) / n
    out = g_ref[...] * zc * jax.lax.rsqrt(v + 1e-5) + be_ref[...]
    if relu:
        out = jnp.maximum(out, 0.0)
    o_ref[...] = out


def _bn_relu(z, g, be, relu):
    n, d = z.shape
    blk = 128
    return pl.pallas_call(
        functools.partial(_bn_relu_kernel, relu=relu),
        grid=(d // blk,),
        in_specs=[
            pl.BlockSpec((n, blk), lambda i: (0, i)),
            pl.BlockSpec((1, blk), lambda i: (0, i)),
            pl.BlockSpec((1, blk), lambda i: (0, i)),
        ],
        out_specs=pl.BlockSpec((n, blk), lambda i: (0, i)),
        out_shape=jax.ShapeDtypeStruct((n, d), jnp.float32),
    )(z, g.reshape(1, -1), be.reshape(1, -1))


def _deg_sc(colp1, ewp1, n_pad):
    e_pad = colp1.shape[0]
    ew_t = e_pad // _NW
    nps = n_pad // _NS
    mesh = plsc.VectorSubcoreMesh(core_axis_name="c", subcore_axis_name="s")

    @functools.partial(
        pl.kernel,
        mesh=mesh,
        out_type=jax.ShapeDtypeStruct((_NC, n_pad), jnp.float32),
        compiler_params=_SC_CP,
        scratch_types=[
            pltpu.VMEM((ew_t,), jnp.int32),
            pltpu.VMEM((ew_t,), jnp.float32),
            pltpu.VMEM((n_pad,), jnp.float32),
            pltpu.VMEM((_NS * nps,), jnp.float32),
            pltpu.VMEM((nps,), jnp.float32),
            pltpu.VMEM_SHARED((_NS, n_pad), jnp.float32),
        ],
    )
    def deg_kernel(col_hbm, ew_hbm, out_hbm, colv, ewv, acc, rbuf, obuf, part):
        ci = lax.axis_index("c")
        si = lax.axis_index("s")
        wid = ci * _NS + si

        @pl.loop(0, n_pad // _L)
        def _(z):
            acc[pl.ds(z * _L, _L)] = jnp.zeros((_L,), jnp.float32)

        pltpu.sync_copy(col_hbm.at[pl.ds(wid * ew_t, ew_t)], colv)
        pltpu.sync_copy(ew_hbm.at[pl.ds(wid * ew_t, ew_t)], ewv)

        @pl.loop(0, ew_t // _L)
        def _(t):
            idx = colv[pl.ds(t * _L, _L)]
            w = ewv[pl.ds(t * _L, _L)]
            plsc.addupdate_scatter(acc, [idx], w)

        pltpu.sync_copy(acc, part.at[si])
        plsc.subcore_barrier()

        for l in range(_NS):
            pltpu.sync_copy(part.at[l, pl.ds(si * nps, nps)],
                            rbuf.at[pl.ds(l * nps, nps)])

        @pl.loop(0, nps // _L)
        def _(m):
            s = jnp.zeros((_L,), jnp.float32)
            for l in range(_NS):
                s = s + rbuf[pl.ds(l * nps + m * _L, _L)]
            obuf[pl.ds(m * _L, _L)] = s

        pltpu.sync_copy(obuf, out_hbm.at[ci, pl.ds(si * nps, nps)])

    return deg_kernel(colp1, ewp1)


def _agg_sc(y3, rowp2, colp2, ewp1, n_pad):
    c_chunks = y3.shape[0]
    cw = y3.shape[2]
    e_pad = ewp1.shape[0]
    blocks_w = e_pad // 64 // _NW
    ew_t = blocks_w * 64
    gsz = 16
    groups = blocks_w // gsz
    nps = n_pad // _NS
    nzb = 16
    mesh = plsc.VectorSubcoreMesh(core_axis_name="c", subcore_axis_name="s")

    @functools.partial(
        pl.kernel,
        mesh=mesh,
        out_type=jax.ShapeDtypeStruct((_NC, c_chunks, n_pad, cw), jnp.float32),
        compiler_params=_SC_CP,
        scratch_types=[
            pltpu.VMEM((gsz, 64), jnp.int32),
            pltpu.VMEM((gsz, 64), jnp.int32),
            pltpu.VMEM((gsz * 64,), jnp.float32),
            pltpu.VMEM((64, cw), jnp.float32),
            pltpu.VMEM((64, cw), jnp.float32),
            pltpu.VMEM((64, cw), jnp.float32),
            pltpu.VMEM((64, cw), jnp.float32),
            pltpu.VMEM((nzb, cw), jnp.float32),
            pltpu.VMEM_SHARED((n_pad, cw), jnp.float32),
            pltpu.SemaphoreType.DMA,
            pltpu.SemaphoreType.DMA,
            pltpu.SemaphoreType.DMA,
            pltpu.SemaphoreType.DMA,
        ],
    )
    def agg_kernel(y_hbm, row_hbm, col_hbm, ew_hbm, out_hbm,
                   rowv, colv, ewv, ga, gb, sa, sb, zbuf, acc,
                   sga, sgb, ssa, ssb):
        ci = lax.axis_index("c")
        si = lax.axis_index("s")
        wid = ci * _NS + si

        @pl.loop(0, nzb)
        def _(r):
            for k in range(cw // _L):
                zbuf[r, pl.ds(k * _L, _L)] = jnp.zeros((_L,), jnp.float32)

        def scale(jj, g, s):
            @plsc.parallel_loop(0, 64, 1, unroll=4)
            def _(e):
                wv = ewv[pl.ds(jj * 64 + (e // _L) * _L, _L)]
                spl = wv.at[jnp.full((_L,), e % _L, jnp.int32)].get(
                    mode="promise_in_bounds")
                for k in range(cw // _L):
                    s[e, pl.ds(k * _L, _L)] = g[e, pl.ds(k * _L, _L)] * spl

        for c in range(c_chunks):
            for b in range(nps // nzb):
                pltpu.sync_copy(zbuf, acc.at[pl.ds(si * nps + b * nzb, nzb)])
            plsc.subcore_barrier()
            yc = y_hbm.at[c]

            @pl.loop(0, groups)
            def _(jg):
                pltpu.sync_copy(
                    row_hbm.at[pl.ds(wid * blocks_w + jg * gsz, gsz)], rowv)
                pltpu.sync_copy(
                    col_hbm.at[pl.ds(wid * blocks_w + jg * gsz, gsz)], colv)
                pltpu.sync_copy(
                    ew_hbm.at[pl.ds(wid * ew_t + jg * gsz * 64, gsz * 64)], ewv)

                pltpu.async_copy(yc.at[rowv.at[0]], ga, sga)
                pltpu.async_copy(yc.at[rowv.at[1]], gb, sgb)

                @pl.loop(0, gsz // 2)
                def _(p):
                    for (dj, g, s, sg, ss) in ((0, ga, sa, sga, ssa),
                                               (1, gb, sb, sgb, ssb)):
                        jj = p * 2 + dj
                        pltpu.make_async_copy(yc.at[rowv.at[jj]], g, sg).wait()

                        @pl.when(p > 0)
                        def _():
                            pltpu.make_async_copy(
                                s, acc.at[colv.at[jj - 2]], ss).wait()

                        scale(jj, g, s)
                        pltpu.async_copy(s, acc.at[colv.at[jj]], ss, add=True)

                        @pl.when(p < gsz // 2 - 1)
                        def _():
                            pltpu.async_copy(yc.at[rowv.at[jj + 2]], g, sg)

                pltpu.make_async_copy(sa, acc.at[colv.at[gsz - 2]], ssa).wait()
                pltpu.make_async_copy(sb, acc.at[colv.at[gsz - 1]], ssb).wait()

            plsc.subcore_barrier()
            pltpu.sync_copy(acc.at[pl.ds(si * nps, nps)],
                            out_hbm.at[ci, c, pl.ds(si * nps, nps)])
            plsc.subcore_barrier()

    return agg_kernel(y3, rowp2, colp2, ewp1)


def _dis_kernel(degp_ref, o_ref):
    deg = degp_ref[0] + degp_ref[1] + 1.0
    o_ref[...] = jax.lax.rsqrt(deg)


def _dis(degp):
    return pl.pallas_call(
        _dis_kernel,
        out_shape=jax.ShapeDtypeStruct(degp.shape[1:], jnp.float32),
    )(degp)


def kernel(x, edge_index, edge_weight, batch, params):
    n = x.shape[0]
    e = edge_weight.shape[0]
    epad = ((e + _NW * 1024 - 1) // (_NW * 1024)) * (_NW * 1024)
    npad = epad - e
    pad_idx = (jnp.arange(npad, dtype=jnp.int32) * 97) % n

    row = jnp.concatenate([edge_index[0], pad_idx])
    col = jnp.concatenate([edge_index[1], pad_idx])
    ew = jnp.concatenate([edge_weight, jnp.zeros((npad,), jnp.float32)])
    n_pad = 10240

    degp = _deg_sc(col, ew, n_pad)
    dis = _dis(degp)[:n]
    disv = dis[:, None]

    rowp2 = row.reshape(-1, 64)
    colp2 = col.reshape(-1, 64)

    cw = 128
    h = x
    for i in range(1, 6):
        xw = _mm(h, params[f"w{i}"])
        y = xw * disv
        d = y.shape[1]
        c_chunks = d // cw
        y3 = jnp.pad(y, ((0, n_pad - n), (0, 0))) \
            .reshape(n_pad, c_chunks, cw).transpose(1, 0, 2)
        aggp = _agg_sc(y3, rowp2, colp2, ew, n_pad)
        agg = ((aggp[0] + aggp[1]).transpose(1, 0, 2).reshape(n_pad, d)[:n]
               * disv + (disv * disv) * xw)
        z = agg + params[f"b{i}"]
        h = _bn_relu(z, params[f"g{i}"], params[f"be{i}"], relu=(i < 5))

    sums = jax.ops.segment_sum(h, batch, num_segments=64)
    cnt = jax.ops.segment_sum(jnp.ones((n,), h.dtype), batch, num_segments=64)
    pooled = sums / jnp.maximum(cnt, 1.0)[:, None]
    pooled = jax.nn.relu(pooled)
    return _mm(pooled, params["w_fc"]) + params["b_fc"]


# speedup vs baseline: 1.5185x; 1.0636x over previous
import dataclasses
import functools

import jax
import jax.numpy as jnp
from jax import lax
from jax.experimental import pallas as pl
from jax.experimental.pallas import tpu as pltpu
from jax.experimental.pallas import tpu_sc as plsc

_NC, _NS, _L = 2, 16, 16
_NW = _NC * _NS

_SC_CP = pltpu.CompilerParams()
if "needs_layout_passes" in pltpu.CompilerParams.__dataclass_fields__:
    _SC_CP = dataclasses.replace(_SC_CP, needs_layout_passes=False)


def _mm_kernel(h_ref, w_ref, o_ref):
    o_ref[...] = jnp.dot(h_ref[...], w_ref[...], preferred_element_type=jnp.float32)


def _mm(h, w):
    n, d_in = h.shape
    d_out = w.shape[1]
    blk = min(n, 2000)
    return pl.pallas_call(
        _mm_kernel,
        grid=(n // blk,),
        in_specs=[
            pl.BlockSpec((blk, d_in), lambda i: (i, 0)),
            pl.BlockSpec((d_in, d_out), lambda i: (0, 0)),
        ],
        out_specs=pl.BlockSpec((blk, d_out), lambda i: (i, 0)),
        out_shape=jax.ShapeDtypeStruct((n, d_out), jnp.float32),
    )(h, w)


def _bn_relu_kernel(z_ref, g_ref, be_ref, o_ref, *, relu):
    z = z_ref[...]
    n = z.shape[0]
    m = jnp.sum(z, axis=0, keepdims=True) / n
    zc = z - m
    v = jnp.sum(zc * zc, axis=0, keepdims=True) / n
    out = g_ref[...] * zc * jax.lax.rsqrt(v + 1e-5) + be_ref[...]
    if relu:
        out = jnp.maximum(out, 0.0)
    o_ref[...] = out


def _bn_relu(z, g, be, relu):
    n, d = z.shape
    blk = 128
    return pl.pallas_call(
        functools.partial(_bn_relu_kernel, relu=relu),
        grid=(d // blk,),
        in_specs=[
            pl.BlockSpec((n, blk), lambda i: (0, i)),
            pl.BlockSpec((1, blk), lambda i: (0, i)),
            pl.BlockSpec((1, blk), lambda i: (0, i)),
        ],
        out_specs=pl.BlockSpec((n, blk), lambda i: (0, i)),
        out_shape=jax.ShapeDtypeStruct((n, d), jnp.float32),
    )(z, g.reshape(1, -1), be.reshape(1, -1))


def _deg_sc(colp1, ewp1, n_pad):
    e_pad = colp1.shape[0]
    ew_t = e_pad // _NW
    nps = n_pad // _NS
    mesh = plsc.VectorSubcoreMesh(core_axis_name="c", subcore_axis_name="s")

    @functools.partial(
        pl.kernel,
        mesh=mesh,
        out_type=jax.ShapeDtypeStruct((_NC, n_pad), jnp.float32),
        compiler_params=_SC_CP,
        scratch_types=[
            pltpu.VMEM((ew_t,), jnp.int32),
            pltpu.VMEM((ew_t,), jnp.float32),
            pltpu.VMEM((n_pad,), jnp.float32),
            pltpu.VMEM((_NS * nps,), jnp.float32),
            pltpu.VMEM((nps,), jnp.float32),
            pltpu.VMEM_SHARED((_NS, n_pad), jnp.float32),
        ],
    )
    def deg_kernel(col_hbm, ew_hbm, out_hbm, colv, ewv, acc, rbuf, obuf, part):
        ci = lax.axis_index("c")
        si = lax.axis_index("s")
        wid = ci * _NS + si

        @pl.loop(0, n_pad // _L)
        def _(z):
            acc[pl.ds(z * _L, _L)] = jnp.zeros((_L,), jnp.float32)

        pltpu.sync_copy(col_hbm.at[pl.ds(wid * ew_t, ew_t)], colv)
        pltpu.sync_copy(ew_hbm.at[pl.ds(wid * ew_t, ew_t)], ewv)

        @pl.loop(0, ew_t // _L)
        def _(t):
            idx = colv[pl.ds(t * _L, _L)]
            w = ewv[pl.ds(t * _L, _L)]
            plsc.addupdate_scatter(acc, [idx], w)

        pltpu.sync_copy(acc, part.at[si])
        plsc.subcore_barrier()

        for l in range(_NS):
            pltpu.sync_copy(part.at[l, pl.ds(si * nps, nps)],
                            rbuf.at[pl.ds(l * nps, nps)])

        @pl.loop(0, nps // _L)
        def _(m):
            s = jnp.zeros((_L,), jnp.float32)
            for l in range(_NS):
                s = s + rbuf[pl.ds(l * nps + m * _L, _L)]
            obuf[pl.ds(m * _L, _L)] = s

        pltpu.sync_copy(obuf, out_hbm.at[ci, pl.ds(si * nps, nps)])

    return deg_kernel(colp1, ewp1)


def _agg_sc(y3, rowp2, colp2, ewp1, n_pad):
    c_chunks = y3.shape[0]
    cw = y3.shape[2]
    e_pad = ewp1.shape[0]
    blocks_w = e_pad // 64 // _NW
    ew_t = blocks_w * 64
    gsz = 32
    groups = blocks_w // gsz
    nps = n_pad // _NS
    nzb = 16
    mesh = plsc.VectorSubcoreMesh(core_axis_name="c", subcore_axis_name="s")

    @functools.partial(
        pl.kernel,
        mesh=mesh,
        out_type=jax.ShapeDtypeStruct((_NC, c_chunks, n_pad, cw), jnp.float32),
        compiler_params=_SC_CP,
        scratch_types=[
            pltpu.VMEM((gsz, 64), jnp.int32),
            pltpu.VMEM((gsz, 64), jnp.int32),
            pltpu.VMEM((gsz * 64,), jnp.float32),
            pltpu.VMEM((64, cw), jnp.float32),
            pltpu.VMEM((64, cw), jnp.float32),
            pltpu.VMEM((64, cw), jnp.float32),
            pltpu.VMEM((64, cw), jnp.float32),
            pltpu.VMEM((nzb, cw), jnp.float32),
            pltpu.VMEM_SHARED((n_pad, cw), jnp.float32),
            pltpu.SemaphoreType.DMA,
            pltpu.SemaphoreType.DMA,
            pltpu.SemaphoreType.DMA,
            pltpu.SemaphoreType.DMA,
        ],
    )
    def agg_kernel(y_hbm, row_hbm, col_hbm, ew_hbm, out_hbm,
                   rowv, colv, ewv, ga, gb, sa, sb, zbuf, acc,
                   sga, sgb, ssa, ssb):
        ci = lax.axis_index("c")
        si = lax.axis_index("s")
        wid = ci * _NS + si

        @pl.loop(0, nzb)
        def _(r):
            for k in range(cw // _L):
                zbuf[r, pl.ds(k * _L, _L)] = jnp.zeros((_L,), jnp.float32)

        def scale(jj, g, s):
            @plsc.parallel_loop(0, 64, 1, unroll=4)
            def _(e):
                wv = ewv[pl.ds(jj * 64 + (e // _L) * _L, _L)]
                spl = wv.at[jnp.full((_L,), e % _L, jnp.int32)].get(
                    mode="promise_in_bounds")
                for k in range(cw // _L):
                    s[e, pl.ds(k * _L, _L)] = g[e, pl.ds(k * _L, _L)] * spl

        for c in range(c_chunks):
            for b in range(nps // nzb):
                pltpu.sync_copy(zbuf, acc.at[pl.ds(si * nps + b * nzb, nzb)])
            plsc.subcore_barrier()
            yc = y_hbm.at[c]

            @pl.loop(0, groups)
            def _(jg):
                pltpu.sync_copy(
                    row_hbm.at[pl.ds(wid * blocks_w + jg * gsz, gsz)], rowv)
                pltpu.sync_copy(
                    col_hbm.at[pl.ds(wid * blocks_w + jg * gsz, gsz)], colv)
                pltpu.sync_copy(
                    ew_hbm.at[pl.ds(wid * ew_t + jg * gsz * 64, gsz * 64)], ewv)

                pltpu.async_copy(yc.at[rowv.at[0]], ga, sga)
                pltpu.async_copy(yc.at[rowv.at[1]], gb, sgb)

                @pl.loop(0, gsz // 2)
                def _(p):
                    for (dj, g, s, sg, ss) in ((0, ga, sa, sga, ssa),
                                               (1, gb, sb, sgb, ssb)):
                        jj = p * 2 + dj
                        pltpu.make_async_copy(yc.at[rowv.at[jj]], g, sg).wait()

                        @pl.when(p > 0)
                        def _():
                            pltpu.make_async_copy(
                                s, acc.at[colv.at[jj - 2]], ss).wait()

                        scale(jj, g, s)
                        pltpu.async_copy(s, acc.at[colv.at[jj]], ss, add=True)

                        @pl.when(p < gsz // 2 - 1)
                        def _():
                            pltpu.async_copy(yc.at[rowv.at[jj + 2]], g, sg)

                pltpu.make_async_copy(sa, acc.at[colv.at[gsz - 2]], ssa).wait()
                pltpu.make_async_copy(sb, acc.at[colv.at[gsz - 1]], ssb).wait()

            plsc.subcore_barrier()
            pltpu.sync_copy(acc.at[pl.ds(si * nps, nps)],
                            out_hbm.at[ci, c, pl.ds(si * nps, nps)])
            plsc.subcore_barrier()

    return agg_kernel(y3, rowp2, colp2, ewp1)


def _dis_kernel(degp_ref, o_ref):
    deg = degp_ref[0] + degp_ref[1] + 1.0
    o_ref[...] = jax.lax.rsqrt(deg)


def _dis(degp):
    return pl.pallas_call(
        _dis_kernel,
        out_shape=jax.ShapeDtypeStruct(degp.shape[1:], jnp.float32),
    )(degp)


def kernel(x, edge_index, edge_weight, batch, params):
    n = x.shape[0]
    e = edge_weight.shape[0]
    epad = ((e + _NW * 1024 - 1) // (_NW * 1024)) * (_NW * 1024)
    npad = epad - e
    pad_idx = (jnp.arange(npad, dtype=jnp.int32) * 97) % n

    row = jnp.concatenate([edge_index[0], pad_idx])
    col = jnp.concatenate([edge_index[1], pad_idx])
    ew = jnp.concatenate([edge_weight, jnp.zeros((npad,), jnp.float32)])
    n_pad = 10240

    degp = _deg_sc(col, ew, n_pad)
    dis = _dis(degp)[:n]
    disv = dis[:, None]

    rowp2 = row.reshape(-1, 64)
    colp2 = col.reshape(-1, 64)

    cw = 128
    h = x
    for i in range(1, 6):
        xw = _mm(h, params[f"w{i}"])
        y = xw * disv
        d = y.shape[1]
        c_chunks = d // cw
        y3 = jnp.pad(y, ((0, n_pad - n), (0, 0))) \
            .reshape(n_pad, c_chunks, cw).transpose(1, 0, 2)
        aggp = _agg_sc(y3, rowp2, colp2, ew, n_pad)
        agg = ((aggp[0] + aggp[1]).transpose(1, 0, 2).reshape(n_pad, d)[:n]
               * disv + (disv * disv) * xw)
        z = agg + params[f"b{i}"]
        h = _bn_relu(z, params[f"g{i}"], params[f"be{i}"], relu=(i < 5))

    sums = jax.ops.segment_sum(h, batch, num_segments=64)
    cnt = jax.ops.segment_sum(jnp.ones((n,), h.dtype), batch, num_segments=64)
    pooled = sums / jnp.maximum(cnt, 1.0)[:, None]
    pooled = jax.nn.relu(pooled)
    return _mm(pooled, params["w_fc"]) + params["b_fc"]


# speedup vs baseline: 1.5675x; 1.0323x over previous
import dataclasses
import functools

import jax
import jax.numpy as jnp
from jax import lax
from jax.experimental import pallas as pl
from jax.experimental.pallas import tpu as pltpu
from jax.experimental.pallas import tpu_sc as plsc

_NC, _NS, _L = 2, 16, 16
_NW = _NC * _NS

_SC_CP = pltpu.CompilerParams()
if "needs_layout_passes" in pltpu.CompilerParams.__dataclass_fields__:
    _SC_CP = dataclasses.replace(_SC_CP, needs_layout_passes=False)


def _mm_kernel(h_ref, w_ref, o_ref):
    o_ref[...] = jnp.dot(h_ref[...], w_ref[...], preferred_element_type=jnp.float32)


def _mm(h, w):
    n, d_in = h.shape
    d_out = w.shape[1]
    blk = min(n, 2000)
    return pl.pallas_call(
        _mm_kernel,
        grid=(n // blk,),
        in_specs=[
            pl.BlockSpec((blk, d_in), lambda i: (i, 0)),
            pl.BlockSpec((d_in, d_out), lambda i: (0, 0)),
        ],
        out_specs=pl.BlockSpec((blk, d_out), lambda i: (i, 0)),
        out_shape=jax.ShapeDtypeStruct((n, d_out), jnp.float32),
    )(h, w)


def _mmc_kernel(h_ref, w_ref, d_ref, o_ref):
    o_ref[0] = jnp.dot(h_ref[...], w_ref[...],
                       preferred_element_type=jnp.float32) * d_ref[...]


def _mm_chunked(h, w, disv):
    n, d_in = h.shape
    d_out = w.shape[1]
    c_chunks = d_out // 128
    blk = 2000
    return pl.pallas_call(
        _mmc_kernel,
        grid=(n // blk, c_chunks),
        in_specs=[
            pl.BlockSpec((blk, d_in), lambda i, j: (i, 0)),
            pl.BlockSpec((d_in, 128), lambda i, j: (0, j)),
            pl.BlockSpec((blk, 1), lambda i, j: (i, 0)),
        ],
        out_specs=pl.BlockSpec((1, blk, 128), lambda i, j: (j, i, 0)),
        out_shape=jax.ShapeDtypeStruct((c_chunks, n, 128), jnp.float32),
    )(h, w, disv)


def _bn_relu_kernel(aggp_ref, y_ref, d_ref, g_ref, be_ref, o_ref, *, relu, n):
    z = (aggp_ref[0, 0, :n, :] + aggp_ref[1, 0, :n, :] + y_ref[0]) * d_ref[...]
    m = jnp.sum(z, axis=0, keepdims=True) / n
    zc = z - m
    v = jnp.sum(zc * zc, axis=0, keepdims=True) / n
    out = g_ref[...] * zc * jax.lax.rsqrt(v + 1e-5) + be_ref[...]
    if relu:
        out = jnp.maximum(out, 0.0)
    o_ref[...] = out


def _bn_relu(aggp, y3, disv, g, be, relu):
    n_pad2 = aggp.shape[2]
    c_chunks, n, _ = y3.shape
    d = c_chunks * 128
    return pl.pallas_call(
        functools.partial(_bn_relu_kernel, relu=relu, n=n),
        grid=(c_chunks,),
        in_specs=[
            pl.BlockSpec((2, 1, n_pad2, 128), lambda j: (0, j, 0, 0)),
            pl.BlockSpec((1, n, 128), lambda j: (j, 0, 0)),
            pl.BlockSpec((n, 1), lambda j: (0, 0)),
            pl.BlockSpec((1, 128), lambda j: (0, j)),
            pl.BlockSpec((1, 128), lambda j: (0, j)),
        ],
        out_specs=pl.BlockSpec((n, 128), lambda j: (0, j)),
        out_shape=jax.ShapeDtypeStruct((n, d), jnp.float32),
    )(aggp, y3, disv, g.reshape(1, -1), be.reshape(1, -1))


def _deg_sc(colp1, ewp1, n_pad):
    e_pad = colp1.shape[0]
    ew_t = e_pad // _NW
    nps = n_pad // _NS
    mesh = plsc.VectorSubcoreMesh(core_axis_name="c", subcore_axis_name="s")

    @functools.partial(
        pl.kernel,
        mesh=mesh,
        out_type=jax.ShapeDtypeStruct((_NC, n_pad), jnp.float32),
        compiler_params=_SC_CP,
        scratch_types=[
            pltpu.VMEM((ew_t,), jnp.int32),
            pltpu.VMEM((ew_t,), jnp.float32),
            pltpu.VMEM((n_pad,), jnp.float32),
            pltpu.VMEM((_NS * nps,), jnp.float32),
            pltpu.VMEM((nps,), jnp.float32),
            pltpu.VMEM_SHARED((_NS, n_pad), jnp.float32),
        ],
    )
    def deg_kernel(col_hbm, ew_hbm, out_hbm, colv, ewv, acc, rbuf, obuf, part):
        ci = lax.axis_index("c")
        si = lax.axis_index("s")
        wid = ci * _NS + si

        @pl.loop(0, n_pad // _L)
        def _(z):
            acc[pl.ds(z * _L, _L)] = jnp.zeros((_L,), jnp.float32)

        pltpu.sync_copy(col_hbm.at[pl.ds(wid * ew_t, ew_t)], colv)
        pltpu.sync_copy(ew_hbm.at[pl.ds(wid * ew_t, ew_t)], ewv)

        @pl.loop(0, ew_t // _L)
        def _(t):
            idx = colv[pl.ds(t * _L, _L)]
            w = ewv[pl.ds(t * _L, _L)]
            plsc.addupdate_scatter(acc, [idx], w)

        pltpu.sync_copy(acc, part.at[si])
        plsc.subcore_barrier()

        for l in range(_NS):
            pltpu.sync_copy(part.at[l, pl.ds(si * nps, nps)],
                            rbuf.at[pl.ds(l * nps, nps)])

        @pl.loop(0, nps // _L)
        def _(m):
            s = jnp.zeros((_L,), jnp.float32)
            for l in range(_NS):
                s = s + rbuf[pl.ds(l * nps + m * _L, _L)]
            obuf[pl.ds(m * _L, _L)] = s

        pltpu.sync_copy(obuf, out_hbm.at[ci, pl.ds(si * nps, nps)])

    return deg_kernel(colp1, ewp1)


def _agg_sc(y3, rowp2, colp2, ewp1, n_pad):
    c_chunks = y3.shape[0]
    cw = y3.shape[2]
    e_pad = ewp1.shape[0]
    blocks_w = e_pad // 64 // _NW
    ew_t = blocks_w * 64
    gsz = 32
    groups = blocks_w // gsz
    nps = n_pad // _NS
    nzb = 16
    mesh = plsc.VectorSubcoreMesh(core_axis_name="c", subcore_axis_name="s")

    @functools.partial(
        pl.kernel,
        mesh=mesh,
        out_type=jax.ShapeDtypeStruct((_NC, c_chunks, n_pad, cw), jnp.float32),
        compiler_params=_SC_CP,
        scratch_types=[
            pltpu.VMEM((gsz, 64), jnp.int32),
            pltpu.VMEM((gsz, 64), jnp.int32),
            pltpu.VMEM((gsz * 64,), jnp.float32),
            pltpu.VMEM((64, cw), jnp.float32),
            pltpu.VMEM((64, cw), jnp.float32),
            pltpu.VMEM((64, cw), jnp.float32),
            pltpu.VMEM((64, cw), jnp.float32),
            pltpu.VMEM((nzb, cw), jnp.float32),
            pltpu.VMEM_SHARED((n_pad, cw), jnp.float32),
            pltpu.SemaphoreType.DMA,
            pltpu.SemaphoreType.DMA,
            pltpu.SemaphoreType.DMA,
            pltpu.SemaphoreType.DMA,
        ],
    )
    def agg_kernel(y_hbm, row_hbm, col_hbm, ew_hbm, out_hbm,
                   rowv, colv, ewv, ga, gb, sa, sb, zbuf, acc,
                   sga, sgb, ssa, ssb):
        ci = lax.axis_index("c")
        si = lax.axis_index("s")
        wid = ci * _NS + si

        @pl.loop(0, nzb)
        def _(r):
            for k in range(cw // _L):
                zbuf[r, pl.ds(k * _L, _L)] = jnp.zeros((_L,), jnp.float32)

        def scale(jj, g, s):
            @plsc.parallel_loop(0, 64, 1, unroll=4)
            def _(e):
                wv = ewv[pl.ds(jj * 64 + (e // _L) * _L, _L)]
                spl = wv.at[jnp.full((_L,), e % _L, jnp.int32)].get(
                    mode="promise_in_bounds")
                for k in range(cw // _L):
                    s[e, pl.ds(k * _L, _L)] = g[e, pl.ds(k * _L, _L)] * spl

        for c in range(c_chunks):
            for b in range(nps // nzb):
                pltpu.sync_copy(zbuf, acc.at[pl.ds(si * nps + b * nzb, nzb)])
            plsc.subcore_barrier()
            yc = y_hbm.at[c]

            @pl.loop(0, groups)
            def _(jg):
                pltpu.sync_copy(
                    row_hbm.at[pl.ds(wid * blocks_w + jg * gsz, gsz)], rowv)
                pltpu.sync_copy(
                    col_hbm.at[pl.ds(wid * blocks_w + jg * gsz, gsz)], colv)
                pltpu.sync_copy(
                    ew_hbm.at[pl.ds(wid * ew_t + jg * gsz * 64, gsz * 64)], ewv)

                pltpu.async_copy(yc.at[rowv.at[0]], ga, sga)
                pltpu.async_copy(yc.at[rowv.at[1]], gb, sgb)

                @pl.loop(0, gsz // 2)
                def _(p):
                    for (dj, g, s, sg, ss) in ((0, ga, sa, sga, ssa),
                                               (1, gb, sb, sgb, ssb)):
                        jj = p * 2 + dj
                        pltpu.make_async_copy(yc.at[rowv.at[jj]], g, sg).wait()

                        @pl.when(p > 0)
                        def _():
                            pltpu.make_async_copy(
                                s, acc.at[colv.at[jj - 2]], ss).wait()

                        scale(jj, g, s)
                        pltpu.async_copy(s, acc.at[colv.at[jj]], ss, add=True)

                        @pl.when(p < gsz // 2 - 1)
                        def _():
                            pltpu.async_copy(yc.at[rowv.at[jj + 2]], g, sg)

                pltpu.make_async_copy(sa, acc.at[colv.at[gsz - 2]], ssa).wait()
                pltpu.make_async_copy(sb, acc.at[colv.at[gsz - 1]], ssb).wait()

            plsc.subcore_barrier()
            pltpu.sync_copy(acc.at[pl.ds(si * nps, nps)],
                            out_hbm.at[ci, c, pl.ds(si * nps, nps)])
            plsc.subcore_barrier()

    return agg_kernel(y3, rowp2, colp2, ewp1)


def _dis_kernel(degp_ref, o_ref):
    deg = degp_ref[0] + degp_ref[1] + 1.0
    o_ref[...] = jax.lax.rsqrt(deg)


def _dis(degp):
    return pl.pallas_call(
        _dis_kernel,
        out_shape=jax.ShapeDtypeStruct(degp.shape[1:], jnp.float32),
    )(degp)


def kernel(x, edge_index, edge_weight, batch, params):
    n = x.shape[0]
    e = edge_weight.shape[0]
    epad = ((e + _NW * 1024 - 1) // (_NW * 1024)) * (_NW * 1024)
    npad = epad - e
    pad_idx = (jnp.arange(npad, dtype=jnp.int32) * 97) % n

    row = jnp.concatenate([edge_index[0], pad_idx])
    col = jnp.concatenate([edge_index[1], pad_idx])
    ew = jnp.concatenate([edge_weight, jnp.zeros((npad,), jnp.float32)])
    n_pad = 10240

    degp = _deg_sc(col, ew, n_pad)
    dis = _dis(degp)[:n]
    disv = dis[:, None]

    rowp2 = row.reshape(-1, 64)
    colp2 = col.reshape(-1, 64)

    h = x
    for i in range(1, 6):
        y3 = _mm_chunked(h, params[f"w{i}"], disv)
        aggp = _agg_sc(y3, rowp2, colp2, ew, n_pad)
        h = _bn_relu(aggp, y3, disv, params[f"g{i}"], params[f"be{i}"],
                     relu=(i < 5))

    sums = jax.ops.segment_sum(h, batch, num_segments=64)
    cnt = jax.ops.segment_sum(jnp.ones((n,), h.dtype), batch, num_segments=64)
    pooled = sums / jnp.maximum(cnt, 1.0)[:, None]
    pooled = jax.nn.relu(pooled)
    return _mm(pooled, params["w_fc"]) + params["b_fc"]
